# baseline (speedup 1.0000x reference)
import jax
import jax.numpy as jnp
from jax.experimental import pallas as pl
from jax.experimental.pallas import tpu as pltpu

_T = 8
_R = 16
_TR = _T * _R
_SCALING = 32.0 / 16.0
_MBLK = 1024
_WCHUNK = 512
_KCAT = 2048 + _TR


def _fused_body(x_ref, tid_ref, w_hbm, b_ref, a_ref, balt_ref, o_ref,
                rhs_ref, lhs_ref, wc0_ref, wc1_ref, sem0, sem1):
    din = x_ref.shape[1]
    dout = balt_ref.shape[0]

    @pl.when(pl.program_id(0) == 0)
    def _():
        nchunks = dout // _WCHUNK
        bufs = (wc0_ref, wc1_ref)
        sems = (sem0, sem1)
        for c in range(min(2, nchunks)):
            pltpu.make_async_copy(
                w_hbm.at[pl.ds(c * _WCHUNK, _WCHUNK), :],
                bufs[c], sems[c]).start()
        for c in range(nchunks):
            pltpu.make_async_copy(
                w_hbm.at[pl.ds(c * _WCHUNK, _WCHUNK), :],
                bufs[c % 2], sems[c % 2]).wait()
            rhs_ref[pl.ds(c * _WCHUNK, _WCHUNK), :din] = (
                bufs[c % 2][...].astype(jnp.bfloat16))
            if c + 2 < nchunks:
                pltpu.make_async_copy(
                    w_hbm.at[pl.ds((c + 2) * _WCHUNK, _WCHUNK), :],
                    bufs[c % 2], sems[c % 2]).start()
        rhs_ref[:, din:] = balt_ref[...]

    xb = x_ref[...].astype(jnp.bfloat16)
    lhs_ref[:, :din] = xb
    z = jax.lax.dot_general(
        xb, a_ref[...], (((1,), (1,)), ((), ())),
        preferred_element_type=jnp.float32)
    tid = tid_ref[...]
    col = jax.lax.broadcasted_iota(jnp.int32, z.shape, 1)
    onehot = ((col >> 4) == tid).astype(jnp.bfloat16)
    lhs_ref[:, din:] = z.astype(jnp.bfloat16) * onehot
    combined = jax.lax.dot_general(
        lhs_ref[...], rhs_ref[...], (((1,), (1,)), ((), ())),
        preferred_element_type=jnp.float32)
    o_ref[...] = combined + b_ref[...]


def kernel(x, task_ids, W, b, lora_A, lora_B):
    ntok, din = x.shape
    dout = W.shape[0]
    tid2d = task_ids.astype(jnp.int32).reshape(ntok, 1)
    a_cat = lora_A.reshape(_TR, din).astype(jnp.bfloat16)
    b_alt = jnp.transpose(lora_B, (1, 0, 2)).reshape(dout, _TR)
    b_alt = (b_alt * _SCALING).astype(jnp.bfloat16)
    b2d = b.reshape(1, dout)

    grid = (ntok // _MBLK,)
    return pl.pallas_call(
        _fused_body,
        grid=grid,
        in_specs=[
            pl.BlockSpec((_MBLK, din), lambda i: (i, 0)),
            pl.BlockSpec((_MBLK, 1), lambda i: (i, 0)),
            pl.BlockSpec(memory_space=pltpu.MemorySpace.HBM),
            pl.BlockSpec((1, dout), lambda i: (0, 0)),
            pl.BlockSpec((_TR, din), lambda i: (0, 0)),
            pl.BlockSpec((dout, _TR), lambda i: (0, 0)),
        ],
        out_specs=pl.BlockSpec((_MBLK, dout), lambda i: (i, 0)),
        out_shape=jax.ShapeDtypeStruct((ntok, dout), jnp.float32),
        scratch_shapes=[
            pltpu.VMEM((dout, _KCAT), jnp.bfloat16),
            pltpu.VMEM((_MBLK, _KCAT), jnp.bfloat16),
            pltpu.VMEM((_WCHUNK, 2048), jnp.float32),
            pltpu.VMEM((_WCHUNK, 2048), jnp.float32),
            pltpu.SemaphoreType.DMA,
            pltpu.SemaphoreType.DMA,
        ],
    )(x, tid2d, W, b2d, a_cat, b_alt)
